# baseline (speedup 1.0000x reference)
import functools

import numpy as np
import jax
import jax.numpy as jnp
from jax import lax
from jax.experimental import pallas as pl
from jax.experimental.pallas import tpu as pltpu

_R1 = (13, 15, 26, 6)
_R2 = (17, 29, 16, 24)
_KS0 = 0
_KS1 = 42
_KS2 = 0 ^ 42 ^ 0x1BD11BDA


def _rotl(x, d):
    return lax.shift_left(x, np.int32(d)) | lax.shift_right_logical(
        x, np.int32(32 - d))


def _threefry_bits(cnt):
    def rounds(x0, x1, rs):
        for r in rs:
            x0 = x0 + x1
            x1 = _rotl(x1, r)
            x1 = x0 ^ x1
        return x0, x1

    x0 = jnp.zeros_like(cnt)
    x1 = cnt + np.int32(_KS1)
    x0, x1 = rounds(x0, x1, _R1)
    x0 = x0 + np.int32(_KS1)
    x1 = x1 + np.int32((_KS2 + 1) & 0xFFFFFFFF)
    x0, x1 = rounds(x0, x1, _R2)
    x0 = x0 + np.int32(_KS2)
    x1 = x1 + np.int32((_KS0 + 2) & 0xFFFFFFFF)
    x0, x1 = rounds(x0, x1, _R1)
    x0 = x0 + np.int32(_KS0)
    x1 = x1 + np.int32((_KS1 + 3) & 0xFFFFFFFF)
    x0, x1 = rounds(x0, x1, _R2)
    x0 = x0 + np.int32(_KS1)
    x1 = x1 + np.int32((_KS2 + 4) & 0xFFFFFFFF)
    x0, x1 = rounds(x0, x1, _R1)
    x0 = x0 + np.int32(_KS2)
    x1 = x1 + np.int32((_KS0 + 5) & 0xFFFFFFFF)
    return x0 ^ x1


def _body(x_ref, a_ref, logg_ref, ent_ref, cnt_ref,
          gxval, gxidx, gsval, gsidx, gslog, rowsum, entacc,
          *, B, V, BV):
    j = pl.program_id(0)
    nb = pl.num_programs(0)

    @pl.when(j == 0)
    def _init():
        neg = jnp.full((B, 1), -jnp.inf, jnp.float32)
        zf = jnp.zeros((B, 1), jnp.float32)
        zi = jnp.zeros((B, 1), jnp.int32)
        gxval[...] = neg
        gsval[...] = neg
        gxidx[...] = zi
        gsidx[...] = zi
        gslog[...] = zf
        rowsum[...] = zf
        entacc[...] = zf

    xb = x_ref[...]
    lane = lax.broadcasted_iota(jnp.int32, (B, BV), 1)
    col = lane + j * BV
    valid = col < V
    row = lax.broadcasted_iota(jnp.int32, (B, BV), 0)
    cnt = row * np.int32(V) + col

    bits = _threefry_bits(cnt)
    fb = lax.shift_right_logical(bits, np.int32(9)) | np.int32(0x3F800000)
    f = lax.bitcast_convert_type(fb, jnp.float32) - np.float32(1.0)
    tiny = np.float32(np.finfo(np.float32).tiny)
    u = jnp.maximum(tiny, f * (np.float32(1.0) - tiny) + tiny)
    g = -jnp.log(-jnp.log(u))

    logx = jnp.log(xb)
    NEG = np.float32(-np.inf)
    BIG = np.int32(2**31 - 1)
    s = jnp.where(valid, logx + g, NEG)
    xm = jnp.where(valid, xb, NEG)

    lmax = jnp.max(xm, axis=1, keepdims=True)
    lidx = jnp.min(jnp.where(xm == lmax, col, BIG), axis=1, keepdims=True)
    upd = lmax > gxval[...]
    gxval[...] = jnp.where(upd, lmax, gxval[...])
    gxidx[...] = jnp.where(upd, lidx, gxidx[...])

    smax = jnp.max(s, axis=1, keepdims=True)
    sidx = jnp.min(jnp.where(s == smax, col, BIG), axis=1, keepdims=True)
    slog = jnp.max(jnp.where(col == sidx, logx, NEG), axis=1, keepdims=True)
    upd2 = smax > gsval[...]
    gsval[...] = jnp.where(upd2, smax, gsval[...])
    gsidx[...] = jnp.where(upd2, sidx, gsidx[...])
    gslog[...] = jnp.where(upd2, slog, gslog[...])

    rowsum[...] += jnp.sum(jnp.where(valid, xb, np.float32(0.0)),
                           axis=1, keepdims=True)
    entacc[...] += jnp.sum(jnp.where(valid, xb * logx, np.float32(0.0)),
                           axis=1, keepdims=True)

    @pl.when(j == nb - 1)
    def _fin():
        a_ref[...] = gsidx[...]
        logg_ref[...] = gslog[...] - jnp.log(rowsum[...])
        ent_ref[...] = (-jnp.sum(entacc[...])).reshape(1, 1)
        aT = gsidx[...].reshape(1, B)
        cnt_ref[...] = jnp.sum((gxidx[...] == aT).astype(jnp.int32)).reshape(1, 1)


@functools.partial(jax.jit, static_argnames=())
def kernel(x, global_idxes):
    B, V = x.shape
    BV = 2048
    nb = pl.cdiv(V, BV)
    body = functools.partial(_body, B=B, V=V, BV=BV)
    a2, logg2, ent2, cnt2 = pl.pallas_call(
        body,
        grid=(nb,),
        in_specs=[pl.BlockSpec((B, BV), lambda j: (0, j))],
        out_specs=[
            pl.BlockSpec((B, 1), lambda j: (0, 0)),
            pl.BlockSpec((B, 1), lambda j: (0, 0)),
            pl.BlockSpec((1, 1), lambda j: (0, 0)),
            pl.BlockSpec((1, 1), lambda j: (0, 0)),
        ],
        out_shape=[
            jax.ShapeDtypeStruct((B, 1), jnp.int32),
            jax.ShapeDtypeStruct((B, 1), jnp.float32),
            jax.ShapeDtypeStruct((1, 1), jnp.float32),
            jax.ShapeDtypeStruct((1, 1), jnp.int32),
        ],
        scratch_shapes=[
            pltpu.VMEM((B, 1), jnp.float32),
            pltpu.VMEM((B, 1), jnp.int32),
            pltpu.VMEM((B, 1), jnp.float32),
            pltpu.VMEM((B, 1), jnp.int32),
            pltpu.VMEM((B, 1), jnp.float32),
            pltpu.VMEM((B, 1), jnp.float32),
            pltpu.VMEM((B, 1), jnp.float32),
        ],
        compiler_params=pltpu.CompilerParams(
            dimension_semantics=("arbitrary",)),
    )(x)
    a = a2.reshape(-1)
    log_g = logg2.reshape(-1)
    entropy = ent2[0, 0]
    matches = cnt2[0, 0]
    draws = jnp.asarray(B, jnp.int32)
    return (x, a, entropy, log_g, global_idxes, matches, draws)
